# baseline (speedup 1.0000x reference)
import functools

import jax
import jax.numpy as jnp
import numpy as np
from jax.experimental import pallas as pl
from jax.experimental.pallas import tpu as pltpu

BLK = 64
EPS = 1e-6
_PAR2 = pltpu.CompilerParams(dimension_semantics=("parallel", "parallel"))


def _dot(a, b, trans_b=False):
    dn = (((1,), (1 if trans_b else 0,)), ((), ()))
    return jax.lax.dot_general(a, b, dn, preferred_element_type=jnp.float32)


def _lse(x, axis):
    m = jnp.max(x, axis=axis, keepdims=True)
    return m + jnp.log(jnp.sum(jnp.exp(x - m), axis=axis, keepdims=True))


def _ln(x, g, b):
    mean = jnp.mean(x, axis=-1, keepdims=True)
    xc = x - mean
    var = jnp.mean(xc * xc, axis=-1, keepdims=True)
    return xc * jax.lax.rsqrt(var + EPS) * g + b


def _qkv_kernel(x_ref, g_ref, b_ref, wq_ref, wk_ref, wv_ref,
                q_ref, k_ref, v_ref, *, np_):
    xn = _ln(x_ref[0], g_ref[...], b_ref[...]).astype(jnp.bfloat16)
    q = _dot(xn, wq_ref[...]).astype(jnp.bfloat16)
    k = _dot(xn, wk_ref[...]).astype(jnp.bfloat16)
    v = _dot(xn, wv_ref[...]).astype(jnp.bfloat16)
    w = q.shape[-1] // np_
    for p in range(np_):
        q_ref[0, p] = q[:, p * w:(p + 1) * w]
        k_ref[0, p] = k[:, p * w:(p + 1) * w]
        v_ref[0, p] = v[:, p * w:(p + 1) * w]


def _sinkhorn_perm(ks, sw):
    logits = _dot(ks.astype(jnp.bfloat16), sw)
    la = logits - _lse(logits, axis=1)
    la = la - _lse(la, axis=0)
    return jnp.exp(jnp.clip(la, -1.0, 1.0)).astype(jnp.bfloat16)


def _perm_kernel(sw_ref, kb_ref, vb_ref, outk_ref, outv_ref, *, dh):
    kb = kb_ref[0, 0]
    vb = vb_ref[0, 0]
    w = 2 * dh
    j = jax.lax.broadcasted_iota(jnp.int32, (kb.shape[1], w), 0) % w
    d = jax.lax.broadcasted_iota(jnp.int32, (kb.shape[1], w), 1)
    g = (j == d).astype(jnp.bfloat16)
    ksum = _dot(kb, g)
    sw = sw_ref[...]
    perm0 = _sinkhorn_perm(ksum[:, :dh], sw)
    perm1 = _sinkhorn_perm(ksum[:, dh:], sw)
    lane = jax.lax.broadcasted_iota(jnp.int32, kb.shape, 1)
    is0 = (lane // dh) % 2 == 0
    outk_ref[0, 0] = jnp.where(
        is0, _dot(perm0, kb), _dot(perm1, kb)).astype(jnp.bfloat16)
    outv_ref[0, 0] = jnp.where(
        is0, _dot(perm0, vb), _dot(perm1, vb)).astype(jnp.bfloat16)


def _attn_kernel(q_ref, k_ref, v_ref, sk_ref, sv_ref, o_ref, *,
                 nb, dh, scale):
    CH = 2
    rows, cols = CH * BLK, 2 * CH * BLK
    ri = jax.lax.broadcasted_iota(jnp.int32, (rows, cols), 0) // BLK
    ci = jax.lax.broadcasted_iota(jnp.int32, (rows, cols), 1) // BLK
    bias = jnp.where(ri == (ci % CH), 0.0, -1e30).astype(jnp.float32)
    for ch in range(nb // CH):
        base = ch * rows
        qp = q_ref[0, 0, pl.ds(base, rows), :]
        kp = k_ref[0, 0, pl.ds(base, rows), :]
        vp = v_ref[0, 0, pl.ds(base, rows), :]
        skp = sk_ref[0, 0, pl.ds(base, rows), :]
        svp = sv_ref[0, 0, pl.ds(base, rows), :]
        outs = []
        for h in (0, 1):
            sl = slice(h * dh, (h + 1) * dh)
            q_h = qp[:, sl] * scale
            kc = jnp.concatenate([kp[:, sl], skp[:, sl]], axis=0)
            vc = jnp.concatenate([vp[:, sl], svp[:, sl]], axis=0)
            s = _dot(q_h, kc, trans_b=True) + bias
            m = jnp.max(s, axis=-1, keepdims=True)
            p = jnp.exp(s - m)
            w = p.astype(jnp.bfloat16)
            recip = 1.0 / jnp.sum(p, axis=-1, keepdims=True)
            outs.append(_dot(w, vc) * recip)
        o = jnp.concatenate(outs, axis=1)
        o_ref[0, 0, pl.ds(base, rows), :] = o.astype(jnp.bfloat16)


def _mlp_kernel(x_ref, o_ref, wo_ref, g_ref, b_ref, w1_ref, b1_ref,
                w2_ref, b2_ref, out_ref, *, np_):
    o2 = jnp.concatenate([o_ref[0, p] for p in range(np_)], axis=1)
    x1 = x_ref[0] + _dot(o2, wo_ref[...])
    y = _ln(x1, g_ref[...], b_ref[...]).astype(jnp.bfloat16)
    h = jax.nn.gelu(_dot(y, w1_ref[...]) + b1_ref[...]).astype(jnp.bfloat16)
    out_ref[0] = x1 + _dot(h, w2_ref[...]) + b2_ref[...]


def kernel(inputs, ln1_scale, ln1_bias, Wq, Wk, Wv, sort_kernel, Wo,
           ln2_scale, ln2_bias, W1, b1, W2, b2):
    B, S, D = inputs.shape
    _, H, DH = Wq.shape
    MLP = W1.shape[1]
    NB = S // BLK
    HD = H * DH
    HP = H // 2
    PW = 2 * DH
    TS = min(512, S)
    TS1 = min(1024, S)

    f32 = jnp.float32
    bf16 = jnp.bfloat16
    ln1_scale2 = ln1_scale.reshape(1, D)
    ln1_bias2 = ln1_bias.reshape(1, D)
    ln2_scale2 = ln2_scale.reshape(1, D)
    ln2_bias2 = ln2_bias.reshape(1, D)
    b1_2 = b1.reshape(1, MLP)
    b2_2 = b2.reshape(1, D)
    Wq2 = Wq.reshape(D, HD).astype(bf16)
    Wk2 = Wk.reshape(D, HD).astype(bf16)
    Wv2 = Wv.reshape(D, HD).astype(bf16)
    Wo2 = Wo.reshape(HD, D).astype(bf16)
    sort_kernel_b = sort_kernel.astype(bf16)
    W1b = W1.astype(bf16)
    W2b = W2.astype(bf16)

    qkv_specs = dict(
        grid=(B, S // TS1),
        in_specs=[
            pl.BlockSpec((1, TS1, D), lambda b, s: (b, s, 0)),
            pl.BlockSpec((1, D), lambda b, s: (0, 0)),
            pl.BlockSpec((1, D), lambda b, s: (0, 0)),
            pl.BlockSpec((D, HD), lambda b, s: (0, 0)),
            pl.BlockSpec((D, HD), lambda b, s: (0, 0)),
            pl.BlockSpec((D, HD), lambda b, s: (0, 0)),
        ],
        out_specs=[
            pl.BlockSpec((1, HP, TS1, PW), lambda b, s: (b, 0, s, 0)),
            pl.BlockSpec((1, HP, TS1, PW), lambda b, s: (b, 0, s, 0)),
            pl.BlockSpec((1, HP, TS1, PW), lambda b, s: (b, 0, s, 0)),
        ],
    )
    q_t, k_t, v_t = pl.pallas_call(
        functools.partial(_qkv_kernel, np_=HP),
        out_shape=[
            jax.ShapeDtypeStruct((B, HP, S, PW), bf16),
            jax.ShapeDtypeStruct((B, HP, S, PW), bf16),
            jax.ShapeDtypeStruct((B, HP, S, PW), bf16),
        ],
        compiler_params=_PAR2,
        **qkv_specs,
    )(inputs, ln1_scale2, ln1_bias2, Wq2, Wk2, Wv2)

    k_blk = k_t.reshape(B, HP, NB, BLK * PW)
    v_blk = v_t.reshape(B, HP, NB, BLK * PW)

    perm_specs = dict(
        grid=(B, HP),
        in_specs=[
            pl.BlockSpec((DH, NB), lambda b, p: (0, 0)),
            pl.BlockSpec((1, 1, NB, BLK * PW), lambda b, p: (b, p, 0, 0)),
            pl.BlockSpec((1, 1, NB, BLK * PW), lambda b, p: (b, p, 0, 0)),
        ],
        out_specs=[
            pl.BlockSpec((1, 1, NB, BLK * PW), lambda b, p: (b, p, 0, 0)),
            pl.BlockSpec((1, 1, NB, BLK * PW), lambda b, p: (b, p, 0, 0)),
        ],
    )
    sk_blk, sv_blk = pl.pallas_call(
        functools.partial(_perm_kernel, dh=DH),
        out_shape=[
            jax.ShapeDtypeStruct((B, HP, NB, BLK * PW), bf16),
            jax.ShapeDtypeStruct((B, HP, NB, BLK * PW), bf16),
        ],
        compiler_params=_PAR2,
        **perm_specs,
    )(sort_kernel_b, k_blk, v_blk)

    sk_t = sk_blk.reshape(B, HP, S, PW)
    sv_t = sv_blk.reshape(B, HP, S, PW)

    attn_specs = dict(
        grid=(B, HP),
        in_specs=[pl.BlockSpec((1, 1, S, PW), lambda b, p: (b, p, 0, 0))] * 5,
        out_specs=pl.BlockSpec((1, 1, S, PW), lambda b, p: (b, p, 0, 0)),
    )
    scale = float(1.0 / np.sqrt(DH))
    o_t = pl.pallas_call(
        functools.partial(_attn_kernel, nb=NB, dh=DH, scale=scale),
        out_shape=jax.ShapeDtypeStruct((B, HP, S, PW), bf16),
        compiler_params=_PAR2,
        **attn_specs,
    )(q_t, k_t, v_t, sk_t, sv_t)

    mlp_specs = dict(
        grid=(B, S // TS),
        in_specs=[
            pl.BlockSpec((1, TS, D), lambda b, s: (b, s, 0)),
            pl.BlockSpec((1, HP, TS, PW), lambda b, s: (b, 0, s, 0)),
            pl.BlockSpec((HD, D), lambda b, s: (0, 0)),
            pl.BlockSpec((1, D), lambda b, s: (0, 0)),
            pl.BlockSpec((1, D), lambda b, s: (0, 0)),
            pl.BlockSpec((D, MLP), lambda b, s: (0, 0)),
            pl.BlockSpec((1, MLP), lambda b, s: (0, 0)),
            pl.BlockSpec((MLP, D), lambda b, s: (0, 0)),
            pl.BlockSpec((1, D), lambda b, s: (0, 0)),
        ],
        out_specs=pl.BlockSpec((1, TS, D), lambda b, s: (b, s, 0)),
    )
    out = pl.pallas_call(
        functools.partial(_mlp_kernel, np_=HP),
        out_shape=jax.ShapeDtypeStruct((B, S, D), f32),
        compiler_params=_PAR2,
        **mlp_specs,
    )(inputs, o_t, Wo2, ln2_scale2, ln2_bias2, W1b, b1_2, W2b, b2_2)

    return out


# speedup vs baseline: 1.2774x; 1.2774x over previous
import functools

import jax
import jax.numpy as jnp
import numpy as np
from jax.experimental import pallas as pl
from jax.experimental.pallas import tpu as pltpu

BLK = 64
EPS = 1e-6
_PAR2 = pltpu.CompilerParams(dimension_semantics=("parallel", "parallel"))


def _dot(a, b, trans_b=False):
    dn = (((1,), (1 if trans_b else 0,)), ((), ()))
    return jax.lax.dot_general(a, b, dn, preferred_element_type=jnp.float32)


def _lse(x, axis):
    m = jnp.max(x, axis=axis, keepdims=True)
    return m + jnp.log(jnp.sum(jnp.exp(x - m), axis=axis, keepdims=True))


def _ln(x, g, b):
    mean = jnp.mean(x, axis=-1, keepdims=True)
    xc = x - mean
    var = jnp.mean(xc * xc, axis=-1, keepdims=True)
    return xc * jax.lax.rsqrt(var + EPS) * g + b


def _qkv_kernel(x_ref, g_ref, b_ref, wq_ref, wk_ref, wv_ref,
                q_ref, k_ref, v_ref, *, nh):
    xn = _ln(x_ref[0], g_ref[...], b_ref[...]).astype(jnp.bfloat16)
    q = _dot(xn, wq_ref[...]).astype(jnp.bfloat16)
    k = _dot(xn, wk_ref[...]).astype(jnp.bfloat16)
    v = _dot(xn, wv_ref[...]).astype(jnp.bfloat16)
    dh = q.shape[-1] // nh
    for h in range(nh):
        q_ref[0, h] = q[:, h * dh:(h + 1) * dh]
        k_ref[0, h] = k[:, h * dh:(h + 1) * dh]
        v_ref[0, h] = v[:, h * dh:(h + 1) * dh]


def _perm_kernel(sw_ref, kb_ref, vb_ref, outk_ref, outv_ref, *, dh):
    kb = kb_ref[0, 0]
    j = jax.lax.broadcasted_iota(jnp.int32, (kb.shape[1], dh), 0) % dh
    d = jax.lax.broadcasted_iota(jnp.int32, (kb.shape[1], dh), 1)
    g = (j == d).astype(jnp.bfloat16)
    ksum = _dot(kb, g)
    logits = _dot(ksum.astype(jnp.bfloat16), sw_ref[...])
    la = logits - _lse(logits, axis=1)
    la = la - _lse(la, axis=0)
    perm = jnp.exp(jnp.clip(la, -1.0, 1.0)).astype(jnp.bfloat16)
    outk_ref[0, 0] = _dot(perm, kb).astype(jnp.bfloat16)
    outv_ref[0, 0] = _dot(perm, vb_ref[0, 0]).astype(jnp.bfloat16)


def _attn_kernel(q_ref, k_ref, v_ref, sk_ref, sv_ref, o_ref, *, nb, scale):
    CH = 2
    rows, cols = CH * BLK, 2 * CH * BLK
    ri = jax.lax.broadcasted_iota(jnp.int32, (rows, cols), 0) // BLK
    ci = jax.lax.broadcasted_iota(jnp.int32, (rows, cols), 1) // BLK
    bias = jnp.where(ri == (ci % CH), 0.0, -1e30).astype(jnp.float32)
    for ch in range(nb // CH):
        base = ch * rows
        q_c = q_ref[0, 0, pl.ds(base, rows), :] * scale
        kc = jnp.concatenate(
            [k_ref[0, 0, pl.ds(base, rows), :],
             sk_ref[0, 0, pl.ds(base, rows), :]], axis=0)
        vc = jnp.concatenate(
            [v_ref[0, 0, pl.ds(base, rows), :],
             sv_ref[0, 0, pl.ds(base, rows), :]], axis=0)
        s = _dot(q_c, kc, trans_b=True) + bias
        m = jnp.max(s, axis=-1, keepdims=True)
        p = jnp.exp(s - m)
        w = p.astype(jnp.bfloat16)
        recip = 1.0 / jnp.sum(p, axis=-1, keepdims=True)
        o = _dot(w, vc) * recip
        o_ref[0, 0, pl.ds(base, rows), :] = o.astype(jnp.bfloat16)


def _mlp_kernel(x_ref, o_ref, wo_ref, g_ref, b_ref, w1_ref, b1_ref,
                w2_ref, b2_ref, out_ref, *, nh):
    o2 = jnp.concatenate([o_ref[0, h] for h in range(nh)], axis=1)
    x1 = x_ref[0] + _dot(o2, wo_ref[...])
    y = _ln(x1, g_ref[...], b_ref[...]).astype(jnp.bfloat16)
    h = jax.nn.gelu(_dot(y, w1_ref[...]) + b1_ref[...]).astype(jnp.bfloat16)
    out_ref[0] = x1 + _dot(h, w2_ref[...]) + b2_ref[...]


def kernel(inputs, ln1_scale, ln1_bias, Wq, Wk, Wv, sort_kernel, Wo,
           ln2_scale, ln2_bias, W1, b1, W2, b2):
    B, S, D = inputs.shape
    _, H, DH = Wq.shape
    MLP = W1.shape[1]
    NB = S // BLK
    HD = H * DH
    TS = min(512, S)
    TS1 = min(1024, S)

    f32 = jnp.float32
    bf16 = jnp.bfloat16
    ln1_scale2 = ln1_scale.reshape(1, D)
    ln1_bias2 = ln1_bias.reshape(1, D)
    ln2_scale2 = ln2_scale.reshape(1, D)
    ln2_bias2 = ln2_bias.reshape(1, D)
    b1_2 = b1.reshape(1, MLP)
    b2_2 = b2.reshape(1, D)
    Wq2 = Wq.reshape(D, HD).astype(bf16)
    Wk2 = Wk.reshape(D, HD).astype(bf16)
    Wv2 = Wv.reshape(D, HD).astype(bf16)
    Wo2 = Wo.reshape(HD, D).astype(bf16)
    sort_kernel_b = sort_kernel.astype(bf16)
    W1b = W1.astype(bf16)
    W2b = W2.astype(bf16)

    qkv_specs = dict(
        grid=(B, S // TS1),
        in_specs=[
            pl.BlockSpec((1, TS1, D), lambda b, s: (b, s, 0)),
            pl.BlockSpec((1, D), lambda b, s: (0, 0)),
            pl.BlockSpec((1, D), lambda b, s: (0, 0)),
            pl.BlockSpec((D, HD), lambda b, s: (0, 0)),
            pl.BlockSpec((D, HD), lambda b, s: (0, 0)),
            pl.BlockSpec((D, HD), lambda b, s: (0, 0)),
        ],
        out_specs=[
            pl.BlockSpec((1, H, TS1, DH), lambda b, s: (b, 0, s, 0)),
            pl.BlockSpec((1, H, TS1, DH), lambda b, s: (b, 0, s, 0)),
            pl.BlockSpec((1, H, TS1, DH), lambda b, s: (b, 0, s, 0)),
        ],
    )
    q_t, k_t, v_t = pl.pallas_call(
        functools.partial(_qkv_kernel, nh=H),
        out_shape=[
            jax.ShapeDtypeStruct((B, H, S, DH), bf16),
            jax.ShapeDtypeStruct((B, H, S, DH), bf16),
            jax.ShapeDtypeStruct((B, H, S, DH), bf16),
        ],
        compiler_params=_PAR2,
        **qkv_specs,
    )(inputs, ln1_scale2, ln1_bias2, Wq2, Wk2, Wv2)

    k_blk = k_t.reshape(B, H, NB, BLK * DH)
    v_blk = v_t.reshape(B, H, NB, BLK * DH)

    perm_specs = dict(
        grid=(B, H),
        in_specs=[
            pl.BlockSpec((DH, NB), lambda b, h: (0, 0)),
            pl.BlockSpec((1, 1, NB, BLK * DH), lambda b, h: (b, h, 0, 0)),
            pl.BlockSpec((1, 1, NB, BLK * DH), lambda b, h: (b, h, 0, 0)),
        ],
        out_specs=[
            pl.BlockSpec((1, 1, NB, BLK * DH), lambda b, h: (b, h, 0, 0)),
            pl.BlockSpec((1, 1, NB, BLK * DH), lambda b, h: (b, h, 0, 0)),
        ],
    )
    sk_blk, sv_blk = pl.pallas_call(
        functools.partial(_perm_kernel, dh=DH),
        out_shape=[
            jax.ShapeDtypeStruct((B, H, NB, BLK * DH), bf16),
            jax.ShapeDtypeStruct((B, H, NB, BLK * DH), bf16),
        ],
        compiler_params=_PAR2,
        **perm_specs,
    )(sort_kernel_b, k_blk, v_blk)

    sk_t = sk_blk.reshape(B, H, S, DH)
    sv_t = sv_blk.reshape(B, H, S, DH)

    attn_specs = dict(
        grid=(B, H),
        in_specs=[pl.BlockSpec((1, 1, S, DH), lambda b, h: (b, h, 0, 0))] * 5,
        out_specs=pl.BlockSpec((1, 1, S, DH), lambda b, h: (b, h, 0, 0)),
    )
    scale = float(1.0 / np.sqrt(DH))
    o_t = pl.pallas_call(
        functools.partial(_attn_kernel, nb=NB, scale=scale),
        out_shape=jax.ShapeDtypeStruct((B, H, S, DH), bf16),
        compiler_params=_PAR2,
        **attn_specs,
    )(q_t, k_t, v_t, sk_t, sv_t)

    mlp_specs = dict(
        grid=(B, S // TS),
        in_specs=[
            pl.BlockSpec((1, TS, D), lambda b, s: (b, s, 0)),
            pl.BlockSpec((1, H, TS, DH), lambda b, s: (b, 0, s, 0)),
            pl.BlockSpec((HD, D), lambda b, s: (0, 0)),
            pl.BlockSpec((1, D), lambda b, s: (0, 0)),
            pl.BlockSpec((1, D), lambda b, s: (0, 0)),
            pl.BlockSpec((D, MLP), lambda b, s: (0, 0)),
            pl.BlockSpec((1, MLP), lambda b, s: (0, 0)),
            pl.BlockSpec((MLP, D), lambda b, s: (0, 0)),
            pl.BlockSpec((1, D), lambda b, s: (0, 0)),
        ],
        out_specs=pl.BlockSpec((1, TS, D), lambda b, s: (b, s, 0)),
    )
    out = pl.pallas_call(
        functools.partial(_mlp_kernel, nh=H),
        out_shape=jax.ShapeDtypeStruct((B, S, D), f32),
        compiler_params=_PAR2,
        **mlp_specs,
    )(inputs, o_t, Wo2, ln2_scale2, ln2_bias2, W1b, b1_2, W2b, b2_2)

    return out


# speedup vs baseline: 1.2905x; 1.0102x over previous
import functools

import jax
import jax.numpy as jnp
import numpy as np
from jax.experimental import pallas as pl
from jax.experimental.pallas import tpu as pltpu

BLK = 64
EPS = 1e-6
_PAR2 = pltpu.CompilerParams(dimension_semantics=("parallel", "parallel"))


def _dot(a, b, trans_b=False):
    dn = (((1,), (1 if trans_b else 0,)), ((), ()))
    return jax.lax.dot_general(a, b, dn, preferred_element_type=jnp.float32)


def _lse(x, axis):
    m = jnp.max(x, axis=axis, keepdims=True)
    return m + jnp.log(jnp.sum(jnp.exp(x - m), axis=axis, keepdims=True))


def _ln(x, g, b):
    mean = jnp.mean(x, axis=-1, keepdims=True)
    xc = x - mean
    var = jnp.mean(xc * xc, axis=-1, keepdims=True)
    return xc * jax.lax.rsqrt(var + EPS) * g + b


def _qkv_kernel(x_ref, g_ref, b_ref, wq_ref, wk_ref, wv_ref,
                q_ref, k_ref, v_ref, *, nh):
    xn = _ln(x_ref[0], g_ref[...], b_ref[...]).astype(jnp.bfloat16)
    q = _dot(xn, wq_ref[...]).astype(jnp.bfloat16)
    k = _dot(xn, wk_ref[...]).astype(jnp.bfloat16)
    v = _dot(xn, wv_ref[...]).astype(jnp.bfloat16)
    dh = q.shape[-1] // nh
    for h in range(nh):
        q_ref[0, h] = q[:, h * dh:(h + 1) * dh]
        k_ref[0, h] = k[:, h * dh:(h + 1) * dh]
        v_ref[0, h] = v[:, h * dh:(h + 1) * dh]


def _perm_kernel(sw_ref, kb_ref, vb_ref, outk_ref, outv_ref, *, dh):
    kb = kb_ref[0, 0]
    j = jax.lax.broadcasted_iota(jnp.int32, (kb.shape[1], dh), 0) % dh
    d = jax.lax.broadcasted_iota(jnp.int32, (kb.shape[1], dh), 1)
    g = (j == d).astype(jnp.bfloat16)
    ksum = _dot(kb, g)
    logits = _dot(ksum.astype(jnp.bfloat16), sw_ref[...])
    la = logits - _lse(logits, axis=1)
    la = la - _lse(la, axis=0)
    perm = jnp.exp(jnp.clip(la, -1.0, 1.0)).astype(jnp.bfloat16)
    outk_ref[0, 0] = _dot(perm, kb).astype(jnp.bfloat16)
    outv_ref[0, 0] = _dot(perm, vb_ref[0, 0]).astype(jnp.bfloat16)


def _attn_kernel(q_ref, k_ref, v_ref, sk_ref, sv_ref, o_ref, *, nb, scale):
    CH = 2
    rows, cols = CH * BLK, 2 * CH * BLK
    ri = jax.lax.broadcasted_iota(jnp.int32, (rows, cols), 0) // BLK
    ci = jax.lax.broadcasted_iota(jnp.int32, (rows, cols), 1) // BLK
    bias = jnp.where(ri == (ci % CH), 0.0, -1e30).astype(jnp.float32)
    for ch in range(nb // CH):
        base = ch * rows
        q_c = q_ref[0, 0, pl.ds(base, rows), :] * scale
        kc = jnp.concatenate(
            [k_ref[0, 0, pl.ds(base, rows), :],
             sk_ref[0, 0, pl.ds(base, rows), :]], axis=0)
        vc = jnp.concatenate(
            [v_ref[0, 0, pl.ds(base, rows), :],
             sv_ref[0, 0, pl.ds(base, rows), :]], axis=0)
        s = _dot(q_c, kc, trans_b=True) + bias
        m = jnp.max(s, axis=-1, keepdims=True)
        p = jnp.exp(s - m)
        w = p.astype(jnp.bfloat16)
        recip = 1.0 / jnp.sum(p, axis=-1, keepdims=True)
        o = _dot(w, vc) * recip
        o_ref[0, 0, pl.ds(base, rows), :] = o.astype(jnp.bfloat16)


def _mlp_kernel(x_ref, o_ref, wo_ref, g_ref, b_ref, w1_ref, b1_ref,
                w2_ref, b2_ref, out_ref, *, nh):
    o2 = jnp.concatenate([o_ref[0, h] for h in range(nh)], axis=1)
    x1 = x_ref[0] + _dot(o2, wo_ref[...])
    y = _ln(x1, g_ref[...], b_ref[...]).astype(jnp.bfloat16)
    h = jax.nn.gelu(_dot(y, w1_ref[...]) + b1_ref[...]).astype(jnp.bfloat16)
    out_ref[0] = x1 + _dot(h, w2_ref[...]) + b2_ref[...]


def kernel(inputs, ln1_scale, ln1_bias, Wq, Wk, Wv, sort_kernel, Wo,
           ln2_scale, ln2_bias, W1, b1, W2, b2):
    B, S, D = inputs.shape
    _, H, DH = Wq.shape
    MLP = W1.shape[1]
    NB = S // BLK
    HD = H * DH
    TS = min(1024, S)
    TS1 = min(1024, S)

    f32 = jnp.float32
    bf16 = jnp.bfloat16
    ln1_scale2 = ln1_scale.reshape(1, D)
    ln1_bias2 = ln1_bias.reshape(1, D)
    ln2_scale2 = ln2_scale.reshape(1, D)
    ln2_bias2 = ln2_bias.reshape(1, D)
    b1_2 = b1.reshape(1, MLP)
    b2_2 = b2.reshape(1, D)
    Wq2 = Wq.reshape(D, HD).astype(bf16)
    Wk2 = Wk.reshape(D, HD).astype(bf16)
    Wv2 = Wv.reshape(D, HD).astype(bf16)
    Wo2 = Wo.reshape(HD, D).astype(bf16)
    sort_kernel_b = sort_kernel.astype(bf16)
    W1b = W1.astype(bf16)
    W2b = W2.astype(bf16)

    qkv_specs = dict(
        grid=(B, S // TS1),
        in_specs=[
            pl.BlockSpec((1, TS1, D), lambda b, s: (b, s, 0)),
            pl.BlockSpec((1, D), lambda b, s: (0, 0)),
            pl.BlockSpec((1, D), lambda b, s: (0, 0)),
            pl.BlockSpec((D, HD), lambda b, s: (0, 0)),
            pl.BlockSpec((D, HD), lambda b, s: (0, 0)),
            pl.BlockSpec((D, HD), lambda b, s: (0, 0)),
        ],
        out_specs=[
            pl.BlockSpec((1, H, TS1, DH), lambda b, s: (b, 0, s, 0)),
            pl.BlockSpec((1, H, TS1, DH), lambda b, s: (b, 0, s, 0)),
            pl.BlockSpec((1, H, TS1, DH), lambda b, s: (b, 0, s, 0)),
        ],
    )
    q_t, k_t, v_t = pl.pallas_call(
        functools.partial(_qkv_kernel, nh=H),
        out_shape=[
            jax.ShapeDtypeStruct((B, H, S, DH), bf16),
            jax.ShapeDtypeStruct((B, H, S, DH), bf16),
            jax.ShapeDtypeStruct((B, H, S, DH), bf16),
        ],
        compiler_params=_PAR2,
        **qkv_specs,
    )(inputs, ln1_scale2, ln1_bias2, Wq2, Wk2, Wv2)

    k_blk = k_t.reshape(B, H, NB, BLK * DH)
    v_blk = v_t.reshape(B, H, NB, BLK * DH)

    perm_specs = dict(
        grid=(B, H),
        in_specs=[
            pl.BlockSpec((DH, NB), lambda b, h: (0, 0)),
            pl.BlockSpec((1, 1, NB, BLK * DH), lambda b, h: (b, h, 0, 0)),
            pl.BlockSpec((1, 1, NB, BLK * DH), lambda b, h: (b, h, 0, 0)),
        ],
        out_specs=[
            pl.BlockSpec((1, 1, NB, BLK * DH), lambda b, h: (b, h, 0, 0)),
            pl.BlockSpec((1, 1, NB, BLK * DH), lambda b, h: (b, h, 0, 0)),
        ],
    )
    sk_blk, sv_blk = pl.pallas_call(
        functools.partial(_perm_kernel, dh=DH),
        out_shape=[
            jax.ShapeDtypeStruct((B, H, NB, BLK * DH), bf16),
            jax.ShapeDtypeStruct((B, H, NB, BLK * DH), bf16),
        ],
        compiler_params=_PAR2,
        **perm_specs,
    )(sort_kernel_b, k_blk, v_blk)

    sk_t = sk_blk.reshape(B, H, S, DH)
    sv_t = sv_blk.reshape(B, H, S, DH)

    attn_specs = dict(
        grid=(B, H),
        in_specs=[pl.BlockSpec((1, 1, S, DH), lambda b, h: (b, h, 0, 0))] * 5,
        out_specs=pl.BlockSpec((1, 1, S, DH), lambda b, h: (b, h, 0, 0)),
    )
    scale = float(1.0 / np.sqrt(DH))
    o_t = pl.pallas_call(
        functools.partial(_attn_kernel, nb=NB, scale=scale),
        out_shape=jax.ShapeDtypeStruct((B, H, S, DH), bf16),
        compiler_params=_PAR2,
        **attn_specs,
    )(q_t, k_t, v_t, sk_t, sv_t)

    mlp_specs = dict(
        grid=(B, S // TS),
        in_specs=[
            pl.BlockSpec((1, TS, D), lambda b, s: (b, s, 0)),
            pl.BlockSpec((1, H, TS, DH), lambda b, s: (b, 0, s, 0)),
            pl.BlockSpec((HD, D), lambda b, s: (0, 0)),
            pl.BlockSpec((1, D), lambda b, s: (0, 0)),
            pl.BlockSpec((1, D), lambda b, s: (0, 0)),
            pl.BlockSpec((D, MLP), lambda b, s: (0, 0)),
            pl.BlockSpec((1, MLP), lambda b, s: (0, 0)),
            pl.BlockSpec((MLP, D), lambda b, s: (0, 0)),
            pl.BlockSpec((1, D), lambda b, s: (0, 0)),
        ],
        out_specs=pl.BlockSpec((1, TS, D), lambda b, s: (b, s, 0)),
    )
    out = pl.pallas_call(
        functools.partial(_mlp_kernel, nh=H),
        out_shape=jax.ShapeDtypeStruct((B, S, D), f32),
        compiler_params=_PAR2,
        **mlp_specs,
    )(inputs, o_t, Wo2, ln2_scale2, ln2_bias2, W1b, b1_2, W2b, b2_2)

    return out
